# Initial kernel scaffold; baseline (speedup 1.0000x reference)
import jax
import jax.numpy as jnp
from jax.experimental import pallas as pl


def kernel(x, edge_index, W1, b1, W2, b2, gamma, beta):
    raise NotImplementedError("write your pallas kernel here")


# speedup vs baseline: 5.8857x; 5.8857x over previous
import functools

import jax
import jax.numpy as jnp
from jax import lax
from jax.experimental import pallas as pl
from jax.experimental.pallas import tpu as pltpu
from jax.experimental.pallas import tpu_sc as plsc

NUM_SC = 2
SUBCORES = 16
NUM_TILES = NUM_SC * SUBCORES
CHUNK = 128
LANES = 16
PAD_ROWS = 16
BN_EPS_CONST = 1e-5


def _sc_aggregate(h, src3, dst3, nch):
    NP, D = h.shape
    rows_per_tile = NP // SUBCORES
    mesh = plsc.VectorSubcoreMesh(
        core_axis_name="c", subcore_axis_name="s",
        num_cores=NUM_SC, num_subcores=SUBCORES)

    @functools.partial(
        pl.kernel,
        out_type=jax.ShapeDtypeStruct((NUM_SC, NP, D), jnp.float32),
        mesh=mesh,
        scratch_types=[
            pltpu.VMEM((CHUNK,), jnp.int32),
            pltpu.VMEM((CHUNK,), jnp.int32),
            pltpu.VMEM((CHUNK, D), jnp.float32),
            pltpu.VMEM((128, D), jnp.float32),
            pltpu.VMEM_SHARED((NP, D), jnp.float32),
            pltpu.SemaphoreType.DMA,
        ],
    )
    def agg_kernel(h_hbm, src_hbm, dst_hbm, out_hbm,
                   idx_s, idx_d, rows, zbuf, acc, sem):
        cid = lax.axis_index("c")
        sid = lax.axis_index("s")
        wid = sid * NUM_SC + cid

        @pl.loop(0, 128)
        def _(r):
            @pl.loop(0, D, step=LANES)
            def _(c0):
                zbuf[r, pl.ds(c0, LANES)] = jnp.zeros((LANES,), jnp.float32)

        @pl.loop(0, rows_per_tile, step=128)
        def _(r0):
            pltpu.sync_copy(zbuf, acc.at[pl.ds(sid * rows_per_tile + r0, 128)])

        plsc.subcore_barrier()

        @pl.loop(0, nch)
        def _(k):
            pltpu.sync_copy(src_hbm.at[wid, k], idx_s)
            pltpu.sync_copy(dst_hbm.at[wid, k], idx_d)
            pltpu.async_copy(h_hbm.at[idx_s], rows, sem).wait()
            pltpu.sync_copy(rows, acc.at[idx_d], add=True)

        plsc.subcore_barrier()
        pltpu.sync_copy(
            acc.at[pl.ds(sid * rows_per_tile, rows_per_tile)],
            out_hbm.at[cid, pl.ds(sid * rows_per_tile, rows_per_tile)])

    return agg_kernel(h, src3, dst3)


def _tc_mlp(h, parts, W1, b1, W2, b2, scale, beta, n_valid, block):
    NP, D = h.shape
    grid = NP // block

    def mlp_kernel(h_ref, p_ref, w1_ref, b1_ref, w2_ref, b2_ref,
                   s_ref, t_ref, out_ref, mean_ref):
        i = pl.program_id(0)
        z = h_ref[...] + p_ref[0] + p_ref[1]
        z = lax.dot_general(z, w1_ref[...], (((1,), (0,)), ((), ())),
                            precision=lax.Precision.HIGHEST,
                            preferred_element_type=jnp.float32) + b1_ref[...]
        z = jnp.maximum(z, 0.0)
        z = lax.dot_general(z, w2_ref[...], (((1,), (0,)), ((), ())),
                            precision=lax.Precision.HIGHEST,
                            preferred_element_type=jnp.float32) + b2_ref[...]
        z = z * s_ref[...] + t_ref[...]
        hn = jnp.maximum(z, 0.0)
        out_ref[...] = hn
        rid = i * block + lax.broadcasted_iota(jnp.int32, (block, D), 0)
        part = jnp.sum(jnp.where(rid < n_valid, hn, 0.0), axis=0,
                       keepdims=True)

        @pl.when(i == 0)
        def _():
            mean_ref[...] = jnp.zeros_like(mean_ref)

        mean_ref[...] += part

        @pl.when(i == grid - 1)
        def _():
            mean_ref[...] = mean_ref[...] * (1.0 / n_valid)

    return pl.pallas_call(
        mlp_kernel,
        grid=(grid,),
        in_specs=[
            pl.BlockSpec((block, D), lambda i: (i, 0)),
            pl.BlockSpec((NUM_SC, block, D), lambda i: (0, i, 0)),
            pl.BlockSpec((D, D), lambda i: (0, 0)),
            pl.BlockSpec((1, D), lambda i: (0, 0)),
            pl.BlockSpec((D, D), lambda i: (0, 0)),
            pl.BlockSpec((1, D), lambda i: (0, 0)),
            pl.BlockSpec((1, D), lambda i: (0, 0)),
            pl.BlockSpec((1, D), lambda i: (0, 0)),
        ],
        out_specs=[
            pl.BlockSpec((block, D), lambda i: (i, 0)),
            pl.BlockSpec((1, D), lambda i: (0, 0)),
        ],
        out_shape=[
            jax.ShapeDtypeStruct((NP, D), jnp.float32),
            jax.ShapeDtypeStruct((1, D), jnp.float32),
        ],
    )(h, parts, W1, b1, W2, b2, scale, beta)


def kernel(x, edge_index, W1, b1, W2, b2, gamma, beta):
    N, D = x.shape
    E = edge_index.shape[1]
    L = W1.shape[0]
    block = 512
    NP = -(-(N + PAD_ROWS) // (SUBCORES * 128)) * (SUBCORES * 128)
    assert NP % block == 0

    nch = -(-E // (NUM_TILES * CHUNK))
    e_pad = nch * NUM_TILES * CHUNK - E
    pad_idx = (jnp.arange(e_pad, dtype=jnp.int32) % PAD_ROWS) + N
    src3 = jnp.concatenate([edge_index[0], pad_idx]).reshape(
        NUM_TILES, nch, CHUNK)
    dst3 = jnp.concatenate([edge_index[1], pad_idx]).reshape(
        NUM_TILES, nch, CHUNK)

    h = jnp.zeros((NP, D), jnp.float32).at[:N].set(x.astype(jnp.float32))
    inv_std = 1.0 / jnp.sqrt(1.0 + BN_EPS_CONST)
    scales = (gamma * inv_std).astype(jnp.float32)

    mean = None
    for i in range(L):
        parts = _sc_aggregate(h, src3, dst3, nch)
        h, mean = _tc_mlp(h, parts, W1[i], b1[i].reshape(1, D), W2[i],
                          b2[i].reshape(1, D), scales[i].reshape(1, D),
                          beta[i].reshape(1, D), N, block)
    return mean


# speedup vs baseline: 10.1023x; 1.7164x over previous
import functools

import jax
import jax.numpy as jnp
from jax import lax
from jax.experimental import pallas as pl
from jax.experimental.pallas import tpu as pltpu
from jax.experimental.pallas import tpu_sc as plsc

NUM_SC = 2
SUBCORES = 16
NUM_TILES = NUM_SC * SUBCORES
CHUNK = 128
LANES = 16
PAD_ROWS = 16
BN_EPS_CONST = 1e-5


def _sc_aggregate(h, idx3, nch, n_full, n_extra):
    NP, D = h.shape
    rows_per_tile = NP // SUBCORES
    n_even = (n_full // 2) * 2
    mesh = plsc.VectorSubcoreMesh(
        core_axis_name="c", subcore_axis_name="s",
        num_cores=NUM_SC, num_subcores=SUBCORES)

    @functools.partial(
        pl.kernel,
        out_type=jax.ShapeDtypeStruct((NUM_SC, NP, D), jnp.float32),
        mesh=mesh,
        scratch_types=[
            pltpu.VMEM((2, CHUNK), jnp.int32),
            pltpu.VMEM((2, CHUNK), jnp.int32),
            pltpu.VMEM((CHUNK, D), jnp.float32),
            pltpu.VMEM((CHUNK, D), jnp.float32),
            pltpu.VMEM_SHARED((NP, D), jnp.float32),
            pltpu.SemaphoreType.DMA,
            pltpu.SemaphoreType.DMA,
            pltpu.SemaphoreType.DMA,
            pltpu.SemaphoreType.DMA,
        ],
    )
    def agg_kernel(h_hbm, idx_hbm, out_hbm,
                   islot0, islot1, rows0, rows1, acc,
                   isem0, isem1, sem0, sem1):
        cid = lax.axis_index("c")
        sid = lax.axis_index("s")
        wid = sid * NUM_SC + cid
        n_mine = n_full + jnp.where(wid < n_extra, 1, 0)

        def fire_idx(k, slot, isem):
            pltpu.async_copy(idx_hbm.at[wid, k], slot, isem)

        def wait_idx(slot, isem):
            pltpu.make_async_copy(idx_hbm.at[wid, 0], slot, isem).wait()

        def fire_gather(slot, rows, sem):
            pltpu.async_copy(h_hbm.at[slot.at[0]], rows, sem)

        def wait_gather(slot, rows, sem):
            pltpu.make_async_copy(h_hbm.at[slot.at[0]], rows, sem).wait()

        def scatter(rows, slot):
            pltpu.sync_copy(rows, acc.at[slot.at[1]], add=True)

        @pl.loop(0, CHUNK)
        def _(r):
            @pl.loop(0, D, step=LANES)
            def _(c0):
                rows0[r, pl.ds(c0, LANES)] = jnp.zeros((LANES,), jnp.float32)

        @pl.loop(0, rows_per_tile, step=CHUNK)
        def _(r0):
            pltpu.sync_copy(
                rows0, acc.at[pl.ds(sid * rows_per_tile + r0, CHUNK)])

        plsc.subcore_barrier()

        @pl.when(0 < n_mine)
        def _():
            fire_idx(0, islot0, isem0)

        @pl.when(1 < n_mine)
        def _():
            fire_idx(1, islot1, isem1)

        if n_even > 0:
            wait_idx(islot0, isem0)
            fire_gather(islot0, rows0, sem0)

            @pl.loop(0, n_even, step=2)
            def _(k0):
                wait_idx(islot1, isem1)
                wait_gather(islot0, rows0, sem0)
                fire_gather(islot1, rows1, sem1)
                scatter(rows0, islot0)

                @pl.when(k0 + 2 < n_mine)
                def _():
                    fire_idx(k0 + 2, islot0, isem0)

                wait_gather(islot1, rows1, sem1)

                @pl.when(k0 + 2 < n_even)
                def _():
                    wait_idx(islot0, isem0)
                    fire_gather(islot0, rows0, sem0)

                scatter(rows1, islot1)

                @pl.when(k0 + 3 < n_mine)
                def _():
                    fire_idx(k0 + 3, islot1, isem1)

        for tail, (slot, isem) in ((n_even, (islot0, isem0)),
                                   (n_even + 1, (islot1, isem1))):
            @pl.when(tail < n_mine)
            def _():
                wait_idx(slot, isem)
                fire_gather(slot, rows0, sem0)
                wait_gather(slot, rows0, sem0)
                scatter(rows0, slot)

        plsc.subcore_barrier()
        pltpu.sync_copy(
            acc.at[pl.ds(sid * rows_per_tile, rows_per_tile)],
            out_hbm.at[cid, pl.ds(sid * rows_per_tile, rows_per_tile)])

    return agg_kernel(h, idx3)


def _tc_mlp(h, parts, W1, b1, W2, b2, scale, beta, n_valid, block):
    NP, D = h.shape
    grid = NP // block

    def mlp_kernel(h_ref, p_ref, w1_ref, b1_ref, w2_ref, b2_ref,
                   s_ref, t_ref, out_ref, mean_ref):
        i = pl.program_id(0)
        z = h_ref[...] + p_ref[0] + p_ref[1]
        z = lax.dot_general(z, w1_ref[...], (((1,), (0,)), ((), ())),
                            precision=lax.Precision.HIGHEST,
                            preferred_element_type=jnp.float32) + b1_ref[...]
        z = jnp.maximum(z, 0.0)
        z = lax.dot_general(z, w2_ref[...], (((1,), (0,)), ((), ())),
                            precision=lax.Precision.HIGHEST,
                            preferred_element_type=jnp.float32) + b2_ref[...]
        z = z * s_ref[...] + t_ref[...]
        hn = jnp.maximum(z, 0.0)
        out_ref[...] = hn
        rid = i * block + lax.broadcasted_iota(jnp.int32, (block, D), 0)
        part = jnp.sum(jnp.where(rid < n_valid, hn, 0.0), axis=0,
                       keepdims=True)

        @pl.when(i == 0)
        def _():
            mean_ref[...] = jnp.zeros_like(mean_ref)

        mean_ref[...] += part

        @pl.when(i == grid - 1)
        def _():
            mean_ref[...] = mean_ref[...] * (1.0 / n_valid)

    return pl.pallas_call(
        mlp_kernel,
        grid=(grid,),
        in_specs=[
            pl.BlockSpec((block, D), lambda i: (i, 0)),
            pl.BlockSpec((NUM_SC, block, D), lambda i: (0, i, 0)),
            pl.BlockSpec((D, D), lambda i: (0, 0)),
            pl.BlockSpec((1, D), lambda i: (0, 0)),
            pl.BlockSpec((D, D), lambda i: (0, 0)),
            pl.BlockSpec((1, D), lambda i: (0, 0)),
            pl.BlockSpec((1, D), lambda i: (0, 0)),
            pl.BlockSpec((1, D), lambda i: (0, 0)),
        ],
        out_specs=[
            pl.BlockSpec((block, D), lambda i: (i, 0)),
            pl.BlockSpec((1, D), lambda i: (0, 0)),
        ],
        out_shape=[
            jax.ShapeDtypeStruct((NP, D), jnp.float32),
            jax.ShapeDtypeStruct((1, D), jnp.float32),
        ],
    )(h, parts, W1, b1, W2, b2, scale, beta)


def kernel(x, edge_index, W1, b1, W2, b2, gamma, beta):
    N, D = x.shape
    E = edge_index.shape[1]
    L = W1.shape[0]
    block = 512
    NP = -(-(N + PAD_ROWS) // (SUBCORES * 128)) * (SUBCORES * 128)
    assert NP % block == 0

    src_e, dst_e = edge_index[0], edge_index[1]
    rem = E % CHUNK
    if rem:
        dummy = (jnp.arange(CHUNK - rem, dtype=jnp.int32) % PAD_ROWS) + N
        src_e = jnp.concatenate([src_e, dummy])
        dst_e = jnp.concatenate([dst_e, dummy])
    n_real_chunks = src_e.shape[0] // CHUNK
    nch = -(-n_real_chunks // NUM_TILES)
    n_full = n_real_chunks // NUM_TILES
    n_extra = n_real_chunks - n_full * NUM_TILES
    slot_pad = nch * NUM_TILES * CHUNK - src_e.shape[0]
    zpad = jnp.zeros((slot_pad,), jnp.int32)
    src3 = jnp.concatenate([src_e, zpad]).reshape(
        nch, NUM_TILES, CHUNK).transpose(1, 0, 2)
    dst3 = jnp.concatenate([dst_e, zpad]).reshape(
        nch, NUM_TILES, CHUNK).transpose(1, 0, 2)
    idx3 = jnp.stack([src3, dst3], axis=2)

    h = jnp.zeros((NP, D), jnp.float32).at[:N].set(x.astype(jnp.float32))
    inv_std = 1.0 / jnp.sqrt(1.0 + BN_EPS_CONST)
    scales = (gamma * inv_std).astype(jnp.float32)

    mean = None
    for i in range(L):
        parts = _sc_aggregate(h, idx3, nch, n_full, n_extra)
        h, mean = _tc_mlp(h, parts, W1[i], b1[i].reshape(1, D), W2[i],
                          b2[i].reshape(1, D), scales[i].reshape(1, D),
                          beta[i].reshape(1, D), N, block)
    return mean


# speedup vs baseline: 10.9903x; 1.0879x over previous
import functools

import jax
import jax.numpy as jnp
from jax import lax
from jax.experimental import pallas as pl
from jax.experimental.pallas import tpu as pltpu
from jax.experimental.pallas import tpu_sc as plsc

NUM_SC = 2
SUBCORES = 16
NUM_TILES = NUM_SC * SUBCORES
CHUNK = 128
LANES = 16
PAD_ROWS = 16
BN_EPS_CONST = 1e-5


def _sc_aggregate(h, idx3, nch, n_full, n_extra):
    NP, D = h.shape
    rows_per_tile = NP // SUBCORES
    n_even = (n_full // 2) * 2
    mesh = plsc.VectorSubcoreMesh(
        core_axis_name="c", subcore_axis_name="s",
        num_cores=NUM_SC, num_subcores=SUBCORES)

    @functools.partial(
        pl.kernel,
        out_type=jax.ShapeDtypeStruct((NUM_SC, NP, D), jnp.float32),
        mesh=mesh,
        scratch_types=[
            pltpu.VMEM((2, CHUNK), jnp.int32),
            pltpu.VMEM((2, CHUNK), jnp.int32),
            pltpu.VMEM((CHUNK, D), jnp.float32),
            pltpu.VMEM((CHUNK, D), jnp.float32),
            pltpu.VMEM_SHARED((NP, D), jnp.float32),
            pltpu.SemaphoreType.DMA,
            pltpu.SemaphoreType.DMA,
            pltpu.SemaphoreType.DMA,
            pltpu.SemaphoreType.DMA,
        ],
    )
    def agg_kernel(h_hbm, idx_hbm, out_hbm,
                   islot0, islot1, rows0, rows1, acc,
                   isem0, isem1, sem0, sem1):
        cid = lax.axis_index("c")
        sid = lax.axis_index("s")
        wid = sid * NUM_SC + cid
        n_mine = n_full + jnp.where(wid < n_extra, 1, 0)

        def fire_idx(k, slot, isem):
            pltpu.async_copy(idx_hbm.at[wid, k], slot, isem)

        def wait_idx(slot, isem):
            pltpu.make_async_copy(idx_hbm.at[wid, 0], slot, isem).wait()

        def fire_gather(slot, rows, sem):
            pltpu.async_copy(h_hbm.at[slot.at[0]], rows, sem)

        def wait_gather(slot, rows, sem):
            pltpu.make_async_copy(h_hbm.at[slot.at[0]], rows, sem).wait()

        def scatter(rows, slot):
            pltpu.sync_copy(rows, acc.at[slot.at[1]], add=True)

        @pl.loop(0, CHUNK)
        def _(r):
            @pl.loop(0, D, step=LANES)
            def _(c0):
                rows0[r, pl.ds(c0, LANES)] = jnp.zeros((LANES,), jnp.float32)

        @pl.loop(0, rows_per_tile, step=CHUNK)
        def _(r0):
            pltpu.sync_copy(
                rows0, acc.at[pl.ds(sid * rows_per_tile + r0, CHUNK)])

        plsc.subcore_barrier()

        @pl.when(0 < n_mine)
        def _():
            fire_idx(0, islot0, isem0)

        @pl.when(1 < n_mine)
        def _():
            fire_idx(1, islot1, isem1)

        if n_even > 0:
            wait_idx(islot0, isem0)
            fire_gather(islot0, rows0, sem0)

            @pl.loop(0, n_even, step=2)
            def _(k0):
                wait_idx(islot1, isem1)
                wait_gather(islot0, rows0, sem0)
                fire_gather(islot1, rows1, sem1)
                scatter(rows0, islot0)

                @pl.when(k0 + 2 < n_mine)
                def _():
                    fire_idx(k0 + 2, islot0, isem0)

                wait_gather(islot1, rows1, sem1)

                @pl.when(k0 + 2 < n_even)
                def _():
                    wait_idx(islot0, isem0)
                    fire_gather(islot0, rows0, sem0)

                scatter(rows1, islot1)

                @pl.when(k0 + 3 < n_mine)
                def _():
                    fire_idx(k0 + 3, islot1, isem1)

        for tail, (slot, isem) in ((n_even, (islot0, isem0)),
                                   (n_even + 1, (islot1, isem1))):
            @pl.when(tail < n_mine)
            def _():
                wait_idx(slot, isem)
                fire_gather(slot, rows0, sem0)
                wait_gather(slot, rows0, sem0)
                scatter(rows0, slot)

        plsc.subcore_barrier()
        pltpu.sync_copy(
            acc.at[pl.ds(sid * rows_per_tile, rows_per_tile)],
            out_hbm.at[cid, pl.ds(sid * rows_per_tile, rows_per_tile)])

    return agg_kernel(h, idx3)


def _tc_mlp(h, parts, W1, b1, W2, b2, scale, beta, n_valid, block,
            with_mean):
    NP, D = h.shape
    grid = NP // block

    def body(h_ref, p_ref, w1_ref, b1_ref, w2_ref, b2_ref, s_ref, t_ref,
             out_ref, mean_ref):
        i = pl.program_id(0)
        z = h_ref[...] + p_ref[0] + p_ref[1]
        z = lax.dot_general(
            z.astype(jnp.bfloat16), w1_ref[...].astype(jnp.bfloat16),
            (((1,), (0,)), ((), ())),
            preferred_element_type=jnp.float32) + b1_ref[...]
        z = jnp.maximum(z, 0.0)
        z = lax.dot_general(
            z.astype(jnp.bfloat16), w2_ref[...].astype(jnp.bfloat16),
            (((1,), (0,)), ((), ())),
            preferred_element_type=jnp.float32) + b2_ref[...]
        z = z * s_ref[...] + t_ref[...]
        hn = jnp.maximum(z, 0.0)
        out_ref[...] = hn
        if mean_ref is not None:
            rid = i * block + lax.broadcasted_iota(jnp.int32, (block, D), 0)
            part = jnp.sum(jnp.where(rid < n_valid, hn, 0.0), axis=0,
                           keepdims=True)

            @pl.when(i == 0)
            def _():
                mean_ref[...] = jnp.zeros_like(mean_ref)

            mean_ref[...] += part

            @pl.when(i == grid - 1)
            def _():
                mean_ref[...] = mean_ref[...] * (1.0 / n_valid)

    if with_mean:
        kern = body
        out_specs = [pl.BlockSpec((block, D), lambda i: (i, 0)),
                     pl.BlockSpec((1, D), lambda i: (0, 0))]
        out_shape = [jax.ShapeDtypeStruct((NP, D), jnp.float32),
                     jax.ShapeDtypeStruct((1, D), jnp.float32)]
    else:
        def kern(*refs):
            body(*refs, None)
        out_specs = [pl.BlockSpec((block, D), lambda i: (i, 0))]
        out_shape = [jax.ShapeDtypeStruct((NP, D), jnp.float32)]

    res = pl.pallas_call(
        kern,
        grid=(grid,),
        in_specs=[
            pl.BlockSpec((block, D), lambda i: (i, 0)),
            pl.BlockSpec((NUM_SC, block, D), lambda i: (0, i, 0)),
            pl.BlockSpec((D, D), lambda i: (0, 0)),
            pl.BlockSpec((1, D), lambda i: (0, 0)),
            pl.BlockSpec((D, D), lambda i: (0, 0)),
            pl.BlockSpec((1, D), lambda i: (0, 0)),
            pl.BlockSpec((1, D), lambda i: (0, 0)),
            pl.BlockSpec((1, D), lambda i: (0, 0)),
        ],
        out_specs=out_specs,
        out_shape=out_shape,
    )(h, parts, W1, b1, W2, b2, scale, beta)
    return res if with_mean else (res[0], None)


def kernel(x, edge_index, W1, b1, W2, b2, gamma, beta):
    N, D = x.shape
    E = edge_index.shape[1]
    L = W1.shape[0]
    block = 1024
    NP = -(-(N + PAD_ROWS) // (SUBCORES * 128)) * (SUBCORES * 128)
    assert NP % block == 0

    src_e, dst_e = edge_index[0], edge_index[1]
    rem = E % CHUNK
    if rem:
        dummy = (jnp.arange(CHUNK - rem, dtype=jnp.int32) % PAD_ROWS) + N
        src_e = jnp.concatenate([src_e, dummy])
        dst_e = jnp.concatenate([dst_e, dummy])
    n_real_chunks = src_e.shape[0] // CHUNK
    nch = -(-n_real_chunks // NUM_TILES)
    n_full = n_real_chunks // NUM_TILES
    n_extra = n_real_chunks - n_full * NUM_TILES
    slot_pad = nch * NUM_TILES * CHUNK - src_e.shape[0]
    zpad = jnp.zeros((slot_pad,), jnp.int32)
    src3 = jnp.concatenate([src_e, zpad]).reshape(
        nch, NUM_TILES, CHUNK).transpose(1, 0, 2)
    dst3 = jnp.concatenate([dst_e, zpad]).reshape(
        nch, NUM_TILES, CHUNK).transpose(1, 0, 2)
    idx3 = jnp.stack([src3, dst3], axis=2)

    h = jnp.zeros((NP, D), jnp.float32).at[:N].set(x.astype(jnp.float32))
    inv_std = 1.0 / jnp.sqrt(1.0 + BN_EPS_CONST)
    scales = (gamma * inv_std).astype(jnp.float32)

    mean = None
    for i in range(L):
        parts = _sc_aggregate(h, idx3, nch, n_full, n_extra)
        h, mean = _tc_mlp(h, parts, W1[i], b1[i].reshape(1, D), W2[i],
                          b2[i].reshape(1, D), scales[i].reshape(1, D),
                          beta[i].reshape(1, D), N, block,
                          with_mean=(i == L - 1))
    return mean
